# Initial kernel scaffold; baseline (speedup 1.0000x reference)
import jax
import jax.numpy as jnp
from jax.experimental import pallas as pl


def kernel(x, edge_index, batch, Wl1, bl1, Wr1, Wl2, bl2, Wr2, W_lin1, b_lin1, W_lin2, b_lin2, Wd1, bd1, Wd2, bd2, Wd3, bd3):
    raise NotImplementedError("write your pallas kernel here")


# speedup vs baseline: 7.5938x; 7.5938x over previous
import functools

import jax
import jax.numpy as jnp
from jax import lax
from jax.experimental import pallas as pl
from jax.experimental.pallas import tpu as pltpu
from jax.experimental.pallas import tpu_sc as plsc

N = 10000
E = 320000
F = 128
H1 = 128
H2 = 32
LAT = 32
NU = 64
G = 16

NC = 2
NS = 16
NW = NC * NS
CH = 80
NCH1 = E // NS // CH
NCH2 = E // NW // CH
NPAD = 10240
RPT = NPAD // NS
ZR = 128
HF = F // 2

_f32 = jnp.float32
_MESH = plsc.VectorSubcoreMesh(core_axis_name="c", subcore_axis_name="s")
_SC_PARAMS = pltpu.CompilerParams(use_tc_tiling_on_sc=False)


def _sc1_body(x0_hbm, x1_hbm, src_hbm, dst_hbm, agg_out, cnt_out,
              src_v, dst_v, rows_v, zbuf, ones_v, z16, agg_sh, cnt_sh, sem):
  c = lax.axis_index("c")
  s = lax.axis_index("s")

  pltpu.sync_copy(src_hbm.at[s], src_v)
  pltpu.sync_copy(dst_hbm.at[s], dst_v)

  zrow = jnp.zeros((16,), _f32)
  orow = jnp.ones((16,), _f32)

  @pl.loop(0, ZR)
  def _(i):
    for j in range(HF // 16):
      zbuf[i, pl.ds(j * 16, 16)] = zrow
    z16[i] = zrow

  @pl.loop(0, CH)
  def _(i):
    ones_v[i] = orow

  for k in range(RPT // ZR):
    off = s * RPT + k * ZR
    pltpu.sync_copy(zbuf, agg_sh.at[pl.ds(off, ZR)])
    pltpu.sync_copy(z16, cnt_sh.at[pl.ds(off, ZR)])

  plsc.subcore_barrier()

  def edge_loop(xref, with_cnt):
    @pl.loop(0, NCH1)
    def _(i):
      pltpu.async_copy(xref.at[src_v.at[i]], rows_v, sem).wait()
      pltpu.sync_copy(rows_v, agg_sh.at[dst_v.at[i]], add=True)
      if with_cnt:
        pltpu.sync_copy(ones_v, cnt_sh.at[dst_v.at[i]], add=True)

  @pl.when(c == 0)
  def _():
    edge_loop(x0_hbm, True)

  @pl.when(c == 1)
  def _():
    edge_loop(x1_hbm, False)

  plsc.subcore_barrier()

  row0 = s * RPT
  pltpu.sync_copy(agg_sh.at[pl.ds(row0, RPT)], agg_out.at[c, pl.ds(row0, RPT)])

  @pl.when(c == 0)
  def _():
    pltpu.sync_copy(cnt_sh.at[pl.ds(row0, RPT)], cnt_out.at[pl.ds(row0, RPT)])


_sc_agg1 = pl.kernel(
    _sc1_body,
    out_type=[
        jax.ShapeDtypeStruct((NC, NPAD, HF), _f32),
        jax.ShapeDtypeStruct((NPAD, 16), _f32),
    ],
    mesh=_MESH,
    scratch_types=[
        pltpu.VMEM((NCH1, CH), jnp.int32),
        pltpu.VMEM((NCH1, CH), jnp.int32),
        pltpu.VMEM((CH, HF), _f32),
        pltpu.VMEM((ZR, HF), _f32),
        pltpu.VMEM((CH, 16), _f32),
        pltpu.VMEM((ZR, 16), _f32),
        pltpu.VMEM_SHARED((NPAD, HF), _f32),
        pltpu.VMEM_SHARED((NPAD, 16), _f32),
        pltpu.SemaphoreType.DMA,
    ],
    compiler_params=_SC_PARAMS,
)


def _sc2_body(y_hbm, src_hbm, dst_hbm, agg_out,
              src_v, dst_v, rows_v, zbuf, agg_sh, sem):
  c = lax.axis_index("c")
  s = lax.axis_index("s")
  wid = c * NS + s

  pltpu.sync_copy(src_hbm.at[wid], src_v)
  pltpu.sync_copy(dst_hbm.at[wid], dst_v)

  zrow = jnp.zeros((16,), _f32)

  @pl.loop(0, ZR)
  def _(i):
    for j in range(H2 // 16):
      zbuf[i, pl.ds(j * 16, 16)] = zrow

  for k in range(RPT // ZR):
    pltpu.sync_copy(zbuf, agg_sh.at[pl.ds(s * RPT + k * ZR, ZR)])

  plsc.subcore_barrier()

  @pl.loop(0, NCH2)
  def _(i):
    pltpu.async_copy(y_hbm.at[src_v.at[i]], rows_v, sem).wait()
    pltpu.sync_copy(rows_v, agg_sh.at[dst_v.at[i]], add=True)

  plsc.subcore_barrier()

  row0 = s * RPT
  pltpu.sync_copy(agg_sh.at[pl.ds(row0, RPT)], agg_out.at[c, pl.ds(row0, RPT)])


_sc_agg2 = pl.kernel(
    _sc2_body,
    out_type=[jax.ShapeDtypeStruct((NC, NPAD, H2), _f32)],
    mesh=_MESH,
    scratch_types=[
        pltpu.VMEM((NCH2, CH), jnp.int32),
        pltpu.VMEM((NCH2, CH), jnp.int32),
        pltpu.VMEM((CH, H2), _f32),
        pltpu.VMEM((ZR, H2), _f32),
        pltpu.VMEM_SHARED((NPAD, H2), _f32),
        pltpu.SemaphoreType.DMA,
    ],
    compiler_params=_SC_PARAMS,
)


def _relu(v):
  return jnp.maximum(v, 0.0)


def _leaky(v):
  return jnp.where(v > 0, v, 0.1 * v)


_NB = 2000


def _tc1_body(a_ref, c_ref, x_ref, wl1_ref, bl1_ref, wr1_ref,
              wl2_ref, bl2_ref, wr2_ref, y_ref, r_ref):
  agg = jnp.concatenate([a_ref[0], a_ref[1]], axis=1)
  cnt = c_ref[:, 0:1]
  mean = agg / jnp.maximum(cnt, 1.0)
  h = _relu(
      jnp.dot(mean, wl1_ref[...], preferred_element_type=_f32)
      + bl1_ref[...]
      + jnp.dot(x_ref[...], wr1_ref[...], preferred_element_type=_f32))
  y_ref[...] = jnp.dot(h, wl2_ref[...], preferred_element_type=_f32)
  r_ref[...] = jnp.dot(h, wr2_ref[...], preferred_element_type=_f32) + bl2_ref[...]


_tc1 = pl.pallas_call(
    _tc1_body,
    grid=(N // _NB,),
    in_specs=[
        pl.BlockSpec((NC, _NB, HF), lambda i: (0, i, 0)),
        pl.BlockSpec((_NB, 16), lambda i: (i, 0)),
        pl.BlockSpec((_NB, F), lambda i: (i, 0)),
        pl.BlockSpec((F, H1), lambda i: (0, 0)),
        pl.BlockSpec((1, H1), lambda i: (0, 0)),
        pl.BlockSpec((F, H1), lambda i: (0, 0)),
        pl.BlockSpec((H1, H2), lambda i: (0, 0)),
        pl.BlockSpec((1, H2), lambda i: (0, 0)),
        pl.BlockSpec((H1, H2), lambda i: (0, 0)),
    ],
    out_specs=[
        pl.BlockSpec((_NB, H2), lambda i: (i, 0)),
        pl.BlockSpec((_NB, H2), lambda i: (i, 0)),
    ],
    out_shape=[
        jax.ShapeDtypeStruct((N, H2), _f32),
        jax.ShapeDtypeStruct((N, H2), _f32),
    ],
)


def _tc2_body(a_ref, c_ref, r_ref, b_ref,
              wlin1_ref, blin1_ref, wlin2_ref, blin2_ref,
              wd1_ref, bd1_ref, wd2_ref, bd2_ref, wd3_ref, bd3_ref,
              enc_ref, z_ref):
  agg = a_ref[0, 0:N, :] + a_ref[1, 0:N, :]
  cnt = c_ref[0:N, 0:1]
  h2 = _relu(agg / jnp.maximum(cnt, 1.0) + r_ref[...])
  gids = lax.broadcasted_iota(jnp.int32, (1, G), 1)
  onehot = (b_ref[...] == gids).astype(_f32)
  dn = (((0,), (0,)), ((), ()))
  pooled = lax.dot_general(onehot, h2, dn, preferred_element_type=_f32)
  ones_col = jnp.full((N, 1), 1.0, _f32)
  gcnt = lax.dot_general(onehot, ones_col, dn, preferred_element_type=_f32)
  ge = pooled / jnp.maximum(gcnt, 1.0)
  ge = _relu(jnp.dot(ge, wlin1_ref[...], preferred_element_type=_f32) + blin1_ref[...])
  enc = _leaky(jnp.dot(ge, wlin2_ref[...], preferred_element_type=_f32) + blin2_ref[...])
  z = _leaky(jnp.dot(enc, wd1_ref[...], preferred_element_type=_f32) + bd1_ref[...])
  z = _leaky(jnp.dot(z, wd2_ref[...], preferred_element_type=_f32) + bd2_ref[...])
  z = jnp.dot(z, wd3_ref[...], preferred_element_type=_f32) + bd3_ref[...]
  enc_ref[...] = enc
  z_ref[...] = z


_tc2 = pl.pallas_call(
    _tc2_body,
    out_shape=[
        jax.ShapeDtypeStruct((G, LAT), _f32),
        jax.ShapeDtypeStruct((G, NU), _f32),
    ],
)


@jax.jit
def kernel(x, edge_index, batch, Wl1, bl1, Wr1, Wl2, bl2, Wr2,
           W_lin1, b_lin1, W_lin2, b_lin2, Wd1, bd1, Wd2, bd2, Wd3, bd3):
  src1 = edge_index[0].reshape(NS, NCH1, CH)
  dst1 = edge_index[1].reshape(NS, NCH1, CH)
  src2 = edge_index[0].reshape(NW, NCH2, CH)
  dst2 = edge_index[1].reshape(NW, NCH2, CH)

  agg1p, cnt = _sc_agg1(x[:, :HF], x[:, HF:], src1, dst1)
  y, r = _tc1(agg1p, cnt, x,
              Wl1, bl1.reshape(1, H1), Wr1,
              Wl2, bl2.reshape(1, H2), Wr2)
  (agg2p,) = _sc_agg2(y, src2, dst2)
  encoded, z = _tc2(agg2p, cnt, r, batch.reshape(N, 1),
                    W_lin1, b_lin1.reshape(1, 32),
                    W_lin2, b_lin2.reshape(1, LAT),
                    Wd1, bd1.reshape(1, 32),
                    Wd2, bd2.reshape(1, 32),
                    Wd3, bd3.reshape(1, NU))
  return (encoded, z)


# speedup vs baseline: 16.8659x; 2.2210x over previous
import functools

import jax
import jax.numpy as jnp
from jax import lax
from jax.experimental import pallas as pl
from jax.experimental.pallas import tpu as pltpu
from jax.experimental.pallas import tpu_sc as plsc

N = 10000
E = 320000
F = 128
H1 = 128
H2 = 32
LAT = 32
NU = 64
G = 16

NC = 2
NS = 16
NW = NC * NS
CH = 80
NCH1 = E // NS // CH
NCH2 = E // NW // CH
NPAD = 10240
RPT = NPAD // NS
ZR = 128
HF = F // 2
NBUF = 5

_f32 = jnp.float32
_MESH = plsc.VectorSubcoreMesh(core_axis_name="c", subcore_axis_name="s")
_SC_PARAMS = pltpu.CompilerParams(use_tc_tiling_on_sc=False)


def _sc1_body(x0_hbm, x1_hbm, src_hbm, dst_hbm, agg_out, cnt_out,
              src_v, dst_v, rows_v, zbuf, ones_v, z16, agg_sh, cnt_sh, *sems):
  c = lax.axis_index("c")
  s = lax.axis_index("s")

  pltpu.sync_copy(src_hbm.at[s], src_v)
  pltpu.sync_copy(dst_hbm.at[s], dst_v)

  zrow = jnp.zeros((16,), _f32)
  orow = jnp.ones((16,), _f32)

  @pl.loop(0, ZR)
  def _(i):
    for j in range(HF // 16):
      zbuf[i, pl.ds(j * 16, 16)] = zrow
    z16[i] = zrow

  @pl.loop(0, CH)
  def _(i):
    ones_v[i] = orow

  for k in range(RPT // ZR):
    off = s * RPT + k * ZR
    pltpu.sync_copy(zbuf, agg_sh.at[pl.ds(off, ZR)])
    pltpu.sync_copy(z16, cnt_sh.at[pl.ds(off, ZR)])

  plsc.subcore_barrier()

  def edge_loop(xref, cnt_lo, cnt_hi):
    for b in range(NBUF):
      pltpu.async_copy(xref.at[src_v.at[b]], rows_v.at[b], sems[b])

    @pl.loop(0, NCH1, step=NBUF)
    def _(i):
      for b in range(NBUF):
        pltpu.make_async_copy(xref.at[src_v.at[0]], rows_v.at[b], sems[b]).wait()
        pltpu.sync_copy(rows_v.at[b], agg_sh.at[dst_v.at[i + b]], add=True)

        @pl.when((i + b >= cnt_lo) & (i + b < cnt_hi))
        def _():
          pltpu.sync_copy(ones_v, cnt_sh.at[dst_v.at[i + b]], add=True)

        nxt = i + b + NBUF

        @pl.when(nxt < NCH1)
        def _():
          pltpu.async_copy(xref.at[src_v.at[nxt]], rows_v.at[b], sems[b])

  @pl.when(c == 0)
  def _():
    edge_loop(x0_hbm, 0, NCH1 // 2)

  @pl.when(c == 1)
  def _():
    edge_loop(x1_hbm, NCH1 // 2, NCH1)

  plsc.subcore_barrier()

  row0 = s * RPT
  pltpu.sync_copy(agg_sh.at[pl.ds(row0, RPT)], agg_out.at[c, pl.ds(row0, RPT)])
  pltpu.sync_copy(cnt_sh.at[pl.ds(row0, RPT)], cnt_out.at[c, pl.ds(row0, RPT)])


_sc_agg1 = pl.kernel(
    _sc1_body,
    out_type=[
        jax.ShapeDtypeStruct((NC, NPAD, HF), _f32),
        jax.ShapeDtypeStruct((NC, NPAD, 16), _f32),
    ],
    mesh=_MESH,
    scratch_types=[
        pltpu.VMEM((NCH1, CH), jnp.int32),
        pltpu.VMEM((NCH1, CH), jnp.int32),
        pltpu.VMEM((NBUF, CH, HF), _f32),
        pltpu.VMEM((ZR, HF), _f32),
        pltpu.VMEM((CH, 16), _f32),
        pltpu.VMEM((ZR, 16), _f32),
        pltpu.VMEM_SHARED((NPAD, HF), _f32),
        pltpu.VMEM_SHARED((NPAD, 16), _f32),
    ] + [pltpu.SemaphoreType.DMA] * NBUF,
    compiler_params=_SC_PARAMS,
)


def _sc2_body(y_hbm, src_hbm, dst_hbm, agg_out,
              src_v, dst_v, rows_v, zbuf, agg_sh, *sems):
  c = lax.axis_index("c")
  s = lax.axis_index("s")
  wid = c * NS + s

  pltpu.sync_copy(src_hbm.at[wid], src_v)
  pltpu.sync_copy(dst_hbm.at[wid], dst_v)

  zrow = jnp.zeros((16,), _f32)

  @pl.loop(0, ZR)
  def _(i):
    for j in range(H2 // 16):
      zbuf[i, pl.ds(j * 16, 16)] = zrow

  for k in range(RPT // ZR):
    pltpu.sync_copy(zbuf, agg_sh.at[pl.ds(s * RPT + k * ZR, ZR)])

  plsc.subcore_barrier()

  for b in range(NBUF):
    pltpu.async_copy(y_hbm.at[src_v.at[b]], rows_v.at[b], sems[b])

  @pl.loop(0, NCH2, step=NBUF)
  def _(i):
    for b in range(NBUF):
      pltpu.make_async_copy(y_hbm.at[src_v.at[0]], rows_v.at[b], sems[b]).wait()
      pltpu.sync_copy(rows_v.at[b], agg_sh.at[dst_v.at[i + b]], add=True)

      nxt = i + b + NBUF

      @pl.when(nxt < NCH2)
      def _():
        pltpu.async_copy(y_hbm.at[src_v.at[nxt]], rows_v.at[b], sems[b])

  plsc.subcore_barrier()

  row0 = s * RPT
  pltpu.sync_copy(agg_sh.at[pl.ds(row0, RPT)], agg_out.at[c, pl.ds(row0, RPT)])


_sc_agg2 = pl.kernel(
    _sc2_body,
    out_type=[jax.ShapeDtypeStruct((NC, NPAD, H2), _f32)],
    mesh=_MESH,
    scratch_types=[
        pltpu.VMEM((NCH2, CH), jnp.int32),
        pltpu.VMEM((NCH2, CH), jnp.int32),
        pltpu.VMEM((NBUF, CH, H2), _f32),
        pltpu.VMEM((ZR, H2), _f32),
        pltpu.VMEM_SHARED((NPAD, H2), _f32),
    ] + [pltpu.SemaphoreType.DMA] * NBUF,
    compiler_params=_SC_PARAMS,
)


def _relu(v):
  return jnp.maximum(v, 0.0)


def _leaky(v):
  return jnp.where(v > 0, v, 0.1 * v)


_NB = 2000


def _tc1_body(a_ref, c_ref, x_ref, wl1_ref, bl1_ref, wr1_ref,
              wl2_ref, bl2_ref, wr2_ref, y_ref, r_ref):
  agg = jnp.concatenate([a_ref[0], a_ref[1]], axis=1)
  cnt = c_ref[0][:, 0:1] + c_ref[1][:, 0:1]
  mean = agg / jnp.maximum(cnt, 1.0)
  h = _relu(
      jnp.dot(mean, wl1_ref[...], preferred_element_type=_f32)
      + bl1_ref[...]
      + jnp.dot(x_ref[...], wr1_ref[...], preferred_element_type=_f32))
  y_ref[...] = jnp.dot(h, wl2_ref[...], preferred_element_type=_f32)
  r_ref[...] = jnp.dot(h, wr2_ref[...], preferred_element_type=_f32) + bl2_ref[...]


_tc1 = pl.pallas_call(
    _tc1_body,
    grid=(N // _NB,),
    in_specs=[
        pl.BlockSpec((NC, _NB, HF), lambda i: (0, i, 0)),
        pl.BlockSpec((NC, _NB, 16), lambda i: (0, i, 0)),
        pl.BlockSpec((_NB, F), lambda i: (i, 0)),
        pl.BlockSpec((F, H1), lambda i: (0, 0)),
        pl.BlockSpec((1, H1), lambda i: (0, 0)),
        pl.BlockSpec((F, H1), lambda i: (0, 0)),
        pl.BlockSpec((H1, H2), lambda i: (0, 0)),
        pl.BlockSpec((1, H2), lambda i: (0, 0)),
        pl.BlockSpec((H1, H2), lambda i: (0, 0)),
    ],
    out_specs=[
        pl.BlockSpec((_NB, H2), lambda i: (i, 0)),
        pl.BlockSpec((_NB, H2), lambda i: (i, 0)),
    ],
    out_shape=[
        jax.ShapeDtypeStruct((N, H2), _f32),
        jax.ShapeDtypeStruct((N, H2), _f32),
    ],
)


def _tc2_body(a_ref, c_ref, r_ref, b_ref,
              wlin1_ref, blin1_ref, wlin2_ref, blin2_ref,
              wd1_ref, bd1_ref, wd2_ref, bd2_ref, wd3_ref, bd3_ref,
              enc_ref, z_ref):
  agg = a_ref[0, 0:N, :] + a_ref[1, 0:N, :]
  cnt = c_ref[0, 0:N, 0:1] + c_ref[1, 0:N, 0:1]
  h2 = _relu(agg / jnp.maximum(cnt, 1.0) + r_ref[...])
  gids = lax.broadcasted_iota(jnp.int32, (1, G), 1)
  onehot = (b_ref[...] == gids).astype(_f32)
  dn = (((0,), (0,)), ((), ()))
  pooled = lax.dot_general(onehot, h2, dn, preferred_element_type=_f32)
  ones_col = jnp.full((N, 1), 1.0, _f32)
  gcnt = lax.dot_general(onehot, ones_col, dn, preferred_element_type=_f32)
  ge = pooled / jnp.maximum(gcnt, 1.0)
  ge = _relu(jnp.dot(ge, wlin1_ref[...], preferred_element_type=_f32) + blin1_ref[...])
  enc = _leaky(jnp.dot(ge, wlin2_ref[...], preferred_element_type=_f32) + blin2_ref[...])
  z = _leaky(jnp.dot(enc, wd1_ref[...], preferred_element_type=_f32) + bd1_ref[...])
  z = _leaky(jnp.dot(z, wd2_ref[...], preferred_element_type=_f32) + bd2_ref[...])
  z = jnp.dot(z, wd3_ref[...], preferred_element_type=_f32) + bd3_ref[...]
  enc_ref[...] = enc
  z_ref[...] = z


_tc2 = pl.pallas_call(
    _tc2_body,
    out_shape=[
        jax.ShapeDtypeStruct((G, LAT), _f32),
        jax.ShapeDtypeStruct((G, NU), _f32),
    ],
)


@jax.jit
def kernel(x, edge_index, batch, Wl1, bl1, Wr1, Wl2, bl2, Wr2,
           W_lin1, b_lin1, W_lin2, b_lin2, Wd1, bd1, Wd2, bd2, Wd3, bd3):
  src1 = edge_index[0].reshape(NS, NCH1, CH)
  dst1 = edge_index[1].reshape(NS, NCH1, CH)
  src2 = edge_index[0].reshape(NW, NCH2, CH)
  dst2 = edge_index[1].reshape(NW, NCH2, CH)

  agg1p, cnt = _sc_agg1(x[:, :HF], x[:, HF:], src1, dst1)
  y, r = _tc1(agg1p, cnt, x,
              Wl1, bl1.reshape(1, H1), Wr1,
              Wl2, bl2.reshape(1, H2), Wr2)
  (agg2p,) = _sc_agg2(y, src2, dst2)
  encoded, z = _tc2(agg2p, cnt, r, batch.reshape(N, 1),
                    W_lin1, b_lin1.reshape(1, 32),
                    W_lin2, b_lin2.reshape(1, LAT),
                    Wd1, bd1.reshape(1, 32),
                    Wd2, bd2.reshape(1, 32),
                    Wd3, bd3.reshape(1, NU))
  return (encoded, z)


# speedup vs baseline: 19.5037x; 1.1564x over previous
import functools

import jax
import jax.numpy as jnp
from jax import lax
from jax.experimental import pallas as pl
from jax.experimental.pallas import tpu as pltpu
from jax.experimental.pallas import tpu_sc as plsc

N = 10000
E = 320000
F = 128
H1 = 128
H2 = 32
LAT = 32
NU = 64
G = 16

NC = 2
NS = 16
NW = NC * NS
CH = 80
NCH1 = E // NS // CH
NCH2 = E // NW // CH
NPAD = 10240
RPT = NPAD // NS
ZR = 128
HF = F // 2
NBUF = 5

_f32 = jnp.float32
_bf16 = jnp.bfloat16
_MESH = plsc.VectorSubcoreMesh(core_axis_name="c", subcore_axis_name="s")
_SC_PARAMS = pltpu.CompilerParams(use_tc_tiling_on_sc=False)


def _sc1_body(x0_hbm, x1_hbm, src_hbm, dst_hbm, agg_out, cnt_out,
              src_v, dst_v, rows_v, zbuf, ones_v, z16, agg_sh, cnt_sh, *sems):
  c = lax.axis_index("c")
  s = lax.axis_index("s")

  pltpu.sync_copy(src_hbm.at[s], src_v)
  pltpu.sync_copy(dst_hbm.at[s], dst_v)

  zrow = jnp.zeros((16,), _f32)
  zrow_b = jnp.zeros((32,), _bf16)
  orow = jnp.ones((16,), _f32)

  @pl.loop(0, ZR)
  def _(i):
    for j in range(HF // 32):
      zbuf[i, pl.ds(j * 32, 32)] = zrow_b
    z16[i] = zrow

  @pl.loop(0, CH)
  def _(i):
    ones_v[i] = orow

  for k in range(RPT // ZR):
    off = s * RPT + k * ZR
    pltpu.sync_copy(zbuf, agg_sh.at[pl.ds(off, ZR)])
    pltpu.sync_copy(z16, cnt_sh.at[pl.ds(off, ZR)])

  plsc.subcore_barrier()

  def edge_loop(xref, cnt_lo, cnt_hi):
    for b in range(NBUF):
      pltpu.async_copy(xref.at[src_v.at[b]], rows_v.at[b], sems[b])

    @pl.loop(0, NCH1, step=NBUF)
    def _(i):
      for b in range(NBUF):
        pltpu.make_async_copy(xref.at[src_v.at[0]], rows_v.at[b], sems[b]).wait()
        pltpu.sync_copy(rows_v.at[b], agg_sh.at[dst_v.at[i + b]], add=True)

        @pl.when((i + b >= cnt_lo) & (i + b < cnt_hi))
        def _():
          pltpu.sync_copy(ones_v, cnt_sh.at[dst_v.at[i + b]], add=True)

        nxt = i + b + NBUF

        @pl.when(nxt < NCH1)
        def _():
          pltpu.async_copy(xref.at[src_v.at[nxt]], rows_v.at[b], sems[b])

  @pl.when(c == 0)
  def _():
    edge_loop(x0_hbm, 0, NCH1 // 2)

  @pl.when(c == 1)
  def _():
    edge_loop(x1_hbm, NCH1 // 2, NCH1)

  plsc.subcore_barrier()

  row0 = s * RPT
  pltpu.sync_copy(agg_sh.at[pl.ds(row0, RPT)], agg_out.at[c, pl.ds(row0, RPT)])
  pltpu.sync_copy(cnt_sh.at[pl.ds(row0, RPT)], cnt_out.at[c, pl.ds(row0, RPT)])


_sc_agg1 = pl.kernel(
    _sc1_body,
    out_type=[
        jax.ShapeDtypeStruct((NC, NPAD, HF), _bf16),
        jax.ShapeDtypeStruct((NC, NPAD, 16), _f32),
    ],
    mesh=_MESH,
    scratch_types=[
        pltpu.VMEM((NCH1, CH), jnp.int32),
        pltpu.VMEM((NCH1, CH), jnp.int32),
        pltpu.VMEM((NBUF, CH, HF), _bf16),
        pltpu.VMEM((ZR, HF), _bf16),
        pltpu.VMEM((CH, 16), _f32),
        pltpu.VMEM((ZR, 16), _f32),
        pltpu.VMEM_SHARED((NPAD, HF), _bf16),
        pltpu.VMEM_SHARED((NPAD, 16), _f32),
    ] + [pltpu.SemaphoreType.DMA] * NBUF,
    compiler_params=_SC_PARAMS,
)


def _sc2_body(y_hbm, src_hbm, dst_hbm, agg_out,
              src_v, dst_v, rows_v, zbuf, agg_sh, *sems):
  c = lax.axis_index("c")
  s = lax.axis_index("s")
  wid = c * NS + s

  pltpu.sync_copy(src_hbm.at[wid], src_v)
  pltpu.sync_copy(dst_hbm.at[wid], dst_v)

  zrow_b = jnp.zeros((32,), _bf16)

  @pl.loop(0, ZR)
  def _(i):
    for j in range(H2 // 32):
      zbuf[i, pl.ds(j * 32, 32)] = zrow_b

  for k in range(RPT // ZR):
    pltpu.sync_copy(zbuf, agg_sh.at[pl.ds(s * RPT + k * ZR, ZR)])

  plsc.subcore_barrier()

  for b in range(NBUF):
    pltpu.async_copy(y_hbm.at[src_v.at[b]], rows_v.at[b], sems[b])

  @pl.loop(0, NCH2, step=NBUF)
  def _(i):
    for b in range(NBUF):
      pltpu.make_async_copy(y_hbm.at[src_v.at[0]], rows_v.at[b], sems[b]).wait()
      pltpu.sync_copy(rows_v.at[b], agg_sh.at[dst_v.at[i + b]], add=True)

      nxt = i + b + NBUF

      @pl.when(nxt < NCH2)
      def _():
        pltpu.async_copy(y_hbm.at[src_v.at[nxt]], rows_v.at[b], sems[b])

  plsc.subcore_barrier()

  row0 = s * RPT
  pltpu.sync_copy(agg_sh.at[pl.ds(row0, RPT)], agg_out.at[c, pl.ds(row0, RPT)])


_sc_agg2 = pl.kernel(
    _sc2_body,
    out_type=[jax.ShapeDtypeStruct((NC, NPAD, H2), _bf16)],
    mesh=_MESH,
    scratch_types=[
        pltpu.VMEM((NCH2, CH), jnp.int32),
        pltpu.VMEM((NCH2, CH), jnp.int32),
        pltpu.VMEM((NBUF, CH, H2), _bf16),
        pltpu.VMEM((ZR, H2), _bf16),
        pltpu.VMEM_SHARED((NPAD, H2), _bf16),
    ] + [pltpu.SemaphoreType.DMA] * NBUF,
    compiler_params=_SC_PARAMS,
)


def _relu(v):
  return jnp.maximum(v, 0.0)


def _leaky(v):
  return jnp.where(v > 0, v, 0.1 * v)


_NB = 2000


def _tc1_body(a_ref, c_ref, x_ref, wl1_ref, bl1_ref, wr1_ref,
              wl2_ref, bl2_ref, wr2_ref, y_ref, r_ref):
  agg = jnp.concatenate([a_ref[0], a_ref[1]], axis=1).astype(_f32)
  cnt = c_ref[0][:, 0:1] + c_ref[1][:, 0:1]
  mean = agg / jnp.maximum(cnt, 1.0)
  h = _relu(
      jnp.dot(mean, wl1_ref[...], preferred_element_type=_f32)
      + bl1_ref[...]
      + jnp.dot(x_ref[...], wr1_ref[...], preferred_element_type=_f32))
  y_ref[...] = jnp.dot(h, wl2_ref[...], preferred_element_type=_f32).astype(_bf16)
  r_ref[...] = jnp.dot(h, wr2_ref[...], preferred_element_type=_f32) + bl2_ref[...]


_tc1 = pl.pallas_call(
    _tc1_body,
    grid=(N // _NB,),
    in_specs=[
        pl.BlockSpec((NC, _NB, HF), lambda i: (0, i, 0)),
        pl.BlockSpec((NC, _NB, 16), lambda i: (0, i, 0)),
        pl.BlockSpec((_NB, F), lambda i: (i, 0)),
        pl.BlockSpec((F, H1), lambda i: (0, 0)),
        pl.BlockSpec((1, H1), lambda i: (0, 0)),
        pl.BlockSpec((F, H1), lambda i: (0, 0)),
        pl.BlockSpec((H1, H2), lambda i: (0, 0)),
        pl.BlockSpec((1, H2), lambda i: (0, 0)),
        pl.BlockSpec((H1, H2), lambda i: (0, 0)),
    ],
    out_specs=[
        pl.BlockSpec((_NB, H2), lambda i: (i, 0)),
        pl.BlockSpec((_NB, H2), lambda i: (i, 0)),
    ],
    out_shape=[
        jax.ShapeDtypeStruct((N, H2), _bf16),
        jax.ShapeDtypeStruct((N, H2), _f32),
    ],
)


def _tc2_body(a_ref, c_ref, r_ref, b_ref,
              wlin1_ref, blin1_ref, wlin2_ref, blin2_ref,
              wd1_ref, bd1_ref, wd2_ref, bd2_ref, wd3_ref, bd3_ref,
              enc_ref, z_ref):
  agg = a_ref[0, 0:N, :].astype(_f32) + a_ref[1, 0:N, :].astype(_f32)
  cnt = c_ref[0, 0:N, 0:1] + c_ref[1, 0:N, 0:1]
  h2 = _relu(agg / jnp.maximum(cnt, 1.0) + r_ref[...])
  gids = lax.broadcasted_iota(jnp.int32, (1, G), 1)
  onehot = (b_ref[...] == gids).astype(_f32)
  dn = (((0,), (0,)), ((), ()))
  pooled = lax.dot_general(onehot, h2, dn, preferred_element_type=_f32)
  ones_col = jnp.full((N, 1), 1.0, _f32)
  gcnt = lax.dot_general(onehot, ones_col, dn, preferred_element_type=_f32)
  ge = pooled / jnp.maximum(gcnt, 1.0)
  ge = _relu(jnp.dot(ge, wlin1_ref[...], preferred_element_type=_f32) + blin1_ref[...])
  enc = _leaky(jnp.dot(ge, wlin2_ref[...], preferred_element_type=_f32) + blin2_ref[...])
  z = _leaky(jnp.dot(enc, wd1_ref[...], preferred_element_type=_f32) + bd1_ref[...])
  z = _leaky(jnp.dot(z, wd2_ref[...], preferred_element_type=_f32) + bd2_ref[...])
  z = jnp.dot(z, wd3_ref[...], preferred_element_type=_f32) + bd3_ref[...]
  enc_ref[...] = enc
  z_ref[...] = z


_tc2 = pl.pallas_call(
    _tc2_body,
    out_shape=[
        jax.ShapeDtypeStruct((G, LAT), _f32),
        jax.ShapeDtypeStruct((G, NU), _f32),
    ],
)


@jax.jit
def kernel(x, edge_index, batch, Wl1, bl1, Wr1, Wl2, bl2, Wr2,
           W_lin1, b_lin1, W_lin2, b_lin2, Wd1, bd1, Wd2, bd2, Wd3, bd3):
  src1 = edge_index[0].reshape(NS, NCH1, CH)
  dst1 = edge_index[1].reshape(NS, NCH1, CH)
  src2 = edge_index[0].reshape(NW, NCH2, CH)
  dst2 = edge_index[1].reshape(NW, NCH2, CH)

  xb = x.astype(_bf16)
  agg1p, cnt = _sc_agg1(xb[:, :HF], xb[:, HF:], src1, dst1)
  y, r = _tc1(agg1p, cnt, x,
              Wl1, bl1.reshape(1, H1), Wr1,
              Wl2, bl2.reshape(1, H2), Wr2)
  (agg2p,) = _sc_agg2(y, src2, dst2)
  encoded, z = _tc2(agg2p, cnt, r, batch.reshape(N, 1),
                    W_lin1, b_lin1.reshape(1, 32),
                    W_lin2, b_lin2.reshape(1, LAT),
                    Wd1, bd1.reshape(1, 32),
                    Wd2, bd2.reshape(1, 32),
                    Wd3, bd3.reshape(1, NU))
  return (encoded, z)
